# baseline (speedup 1.0000x reference)
import jax
import jax.numpy as jnp
from jax.experimental import pallas as pl
from jax.experimental.pallas import tpu as pltpu

N_TOTAL = 10000
N_HALF = 5000
EMBED = 16
BLK = 200
NBLK = N_TOTAL // BLK
BLK2 = 1000
NBLK2 = N_TOTAL // BLK2
F8MAX = 448.0


def _pass1_kernel(a_ref, emb_ref, e1_ref, q_ref, rs_ref):
    a = a_ref[...]
    e1_ref[...] = jnp.dot(
        a.astype(jnp.bfloat16), emb_ref[...].astype(jnp.bfloat16),
        preferred_element_type=jnp.float32)
    q_ref[...] = a.astype(jnp.float8_e4m3fn)
    rs_ref[...] = jnp.sum(a, axis=1, keepdims=True)


def _pass23_kernel(q_ref, rs_ref, emb_ref, e1_ref, out_ref, d8, cur, acc, cns):
    p = pl.program_id(0)
    b = pl.program_id(1)
    rows = pl.ds(b * BLK2, BLK2)

    @pl.when(b == 0)
    def _():
        x = jnp.where(p == 0, e1_ref[...], cur[...])
        c = jnp.mean(x, axis=0, keepdims=True)
        d = x - c
        s = jnp.maximum(
            jnp.max(jnp.abs(d), axis=0, keepdims=True) / F8MAX, 1e-30)
        d8[...] = (d / s).astype(jnp.float8_e4m3fn)
        cns[0:1, :] = s
        cns[1:2, :] = c

    m = jnp.dot(q_ref[...], d8[...], preferred_element_type=jnp.float32)
    val = m * cns[0:1, :] + rs_ref[...] * cns[1:2, :]

    @pl.when(p == 0)
    def _():
        cur[rows, :] = val
        acc[rows, :] = emb_ref[rows, :] + e1_ref[rows, :] + val
        out_ref[0, :, :] = val

    @pl.when(p == 1)
    def _():
        out_ref[0, :, :] = (acc[rows, :] + val) * 0.25


def kernel(interaction_matrix, user_embeds, item_embeds):
    embeds = jnp.concatenate([user_embeds, item_embeds], axis=0)
    e1, q, rowsum = pl.pallas_call(
        _pass1_kernel,
        grid=(NBLK,),
        in_specs=[
            pl.BlockSpec((BLK, N_TOTAL), lambda b: (b, 0)),
            pl.BlockSpec((N_TOTAL, EMBED), lambda b: (0, 0)),
        ],
        out_specs=[
            pl.BlockSpec((BLK, EMBED), lambda b: (b, 0)),
            pl.BlockSpec((BLK, N_TOTAL), lambda b: (b, 0)),
            pl.BlockSpec((BLK, 1), lambda b: (b, 0)),
        ],
        out_shape=[
            jax.ShapeDtypeStruct((N_TOTAL, EMBED), jnp.float32),
            jax.ShapeDtypeStruct((N_TOTAL, N_TOTAL), jnp.float8_e4m3fn),
            jax.ShapeDtypeStruct((N_TOTAL, 1), jnp.float32),
        ],
        compiler_params=pltpu.CompilerParams(
            dimension_semantics=("arbitrary",),
        ),
    )(interaction_matrix, embeds)

    out = pl.pallas_call(
        _pass23_kernel,
        grid=(2, NBLK2),
        in_specs=[
            pl.BlockSpec((BLK2, N_TOTAL), lambda p, b: (b, 0)),
            pl.BlockSpec((BLK2, 1), lambda p, b: (b, 0)),
            pl.BlockSpec((N_TOTAL, EMBED), lambda p, b: (0, 0)),
            pl.BlockSpec((N_TOTAL, EMBED), lambda p, b: (0, 0)),
        ],
        out_specs=pl.BlockSpec((1, BLK2, EMBED), lambda p, b: (p, b, 0)),
        out_shape=jax.ShapeDtypeStruct((2, N_TOTAL, EMBED), jnp.float32),
        scratch_shapes=[
            pltpu.VMEM((N_TOTAL, EMBED), jnp.float8_e4m3fn),
            pltpu.VMEM((N_TOTAL, EMBED), jnp.float32),
            pltpu.VMEM((N_TOTAL, EMBED), jnp.float32),
            pltpu.VMEM((2, EMBED), jnp.float32),
        ],
        compiler_params=pltpu.CompilerParams(
            dimension_semantics=("arbitrary", "arbitrary"),
        ),
    )(q, rowsum, embeds, e1)

    all_emb = out[1]
    return (all_emb[:N_HALF], all_emb[N_HALF:])


# speedup vs baseline: 1.0019x; 1.0019x over previous
import jax
import jax.numpy as jnp
from jax.experimental import pallas as pl
from jax.experimental.pallas import tpu as pltpu

N_TOTAL = 10000
N_HALF = 5000
EMBED = 16
AUG = EMBED + 1
BLK = 400
NBLK = N_TOTAL // BLK
BLK2 = 1000
NBLK2 = N_TOTAL // BLK2
F8MAX = 448.0


def _pass1_kernel(a_ref, aug_ref, e1_ref, q_ref):
    a = a_ref[...]
    e1_ref[...] = jnp.dot(
        a.astype(jnp.bfloat16), aug_ref[...].astype(jnp.bfloat16),
        preferred_element_type=jnp.float32)
    q_ref[...] = a.astype(jnp.float8_e4m3fn)


def _pass23_kernel(q_ref, e1_ref, acc0_ref, out_ref, d8, cur, acc, cns):
    p = pl.program_id(0)
    b = pl.program_id(1)
    rows = pl.ds(b * BLK2, BLK2)

    @pl.when(b == 0)
    def _():
        x = jnp.where(p == 0, e1_ref[:, :EMBED], cur[...])
        c = jnp.mean(x, axis=0, keepdims=True)
        d = x - c
        s = jnp.maximum(
            jnp.max(jnp.abs(d), axis=0, keepdims=True) / F8MAX, 1e-30)
        d8[...] = (d / s).astype(jnp.float8_e4m3fn)
        cns[0:1, :] = s
        cns[1:2, :] = c

    m = jnp.dot(q_ref[...], d8[...], preferred_element_type=jnp.float32)
    val = m * cns[0:1, :] + e1_ref[rows, EMBED:AUG] * cns[1:2, :]

    @pl.when(p == 0)
    def _():
        cur[rows, :] = val
        acc[rows, :] = acc0_ref[rows, :] + val
        out_ref[0, :, :] = val

    @pl.when(p == 1)
    def _():
        out_ref[0, :, :] = (acc[rows, :] + val) * 0.25


def kernel(interaction_matrix, user_embeds, item_embeds):
    embeds = jnp.concatenate([user_embeds, item_embeds], axis=0)
    aug = jnp.concatenate(
        [embeds, jnp.ones((N_TOTAL, 1), jnp.float32)], axis=1)
    e1aug, q = pl.pallas_call(
        _pass1_kernel,
        grid=(NBLK,),
        in_specs=[
            pl.BlockSpec((BLK, N_TOTAL), lambda b: (b, 0)),
            pl.BlockSpec((N_TOTAL, AUG), lambda b: (0, 0)),
        ],
        out_specs=[
            pl.BlockSpec((BLK, AUG), lambda b: (b, 0)),
            pl.BlockSpec((BLK, N_TOTAL), lambda b: (b, 0)),
        ],
        out_shape=[
            jax.ShapeDtypeStruct((N_TOTAL, AUG), jnp.float32),
            jax.ShapeDtypeStruct((N_TOTAL, N_TOTAL), jnp.float8_e4m3fn),
        ],
        compiler_params=pltpu.CompilerParams(
            dimension_semantics=("arbitrary",),
        ),
    )(interaction_matrix, aug)

    acc0 = embeds + e1aug[:, :EMBED]
    out = pl.pallas_call(
        _pass23_kernel,
        grid=(2, NBLK2),
        in_specs=[
            pl.BlockSpec((BLK2, N_TOTAL), lambda p, b: (b, 0)),
            pl.BlockSpec((N_TOTAL, AUG), lambda p, b: (0, 0)),
            pl.BlockSpec((N_TOTAL, EMBED), lambda p, b: (0, 0)),
        ],
        out_specs=pl.BlockSpec((1, BLK2, EMBED), lambda p, b: (p, b, 0)),
        out_shape=jax.ShapeDtypeStruct((2, N_TOTAL, EMBED), jnp.float32),
        scratch_shapes=[
            pltpu.VMEM((N_TOTAL, EMBED), jnp.float8_e4m3fn),
            pltpu.VMEM((N_TOTAL, EMBED), jnp.float32),
            pltpu.VMEM((N_TOTAL, EMBED), jnp.float32),
            pltpu.VMEM((2, EMBED), jnp.float32),
        ],
        compiler_params=pltpu.CompilerParams(
            dimension_semantics=("arbitrary", "arbitrary"),
        ),
    )(q, e1aug, acc0)

    all_emb = out[1]
    return (all_emb[:N_HALF], all_emb[N_HALF:])


# speedup vs baseline: 1.0679x; 1.0659x over previous
import jax
import jax.numpy as jnp
from jax.experimental import pallas as pl
from jax.experimental.pallas import tpu as pltpu

N_TOTAL = 10000
N_HALF = 5000
EMBED = 16
BLK = 400
NBLK = N_TOTAL // BLK
BLK2 = 1000
NBLK2 = N_TOTAL // BLK2
F8MAX = 448.0


def _pass1_kernel(a_ref, emb_ref, e1_ref, q_ref, rs_ref):
    a = a_ref[...]
    e1_ref[...] = jnp.dot(
        a.astype(jnp.bfloat16), emb_ref[...].astype(jnp.bfloat16),
        preferred_element_type=jnp.float32)
    q_ref[...] = a.astype(jnp.float8_e4m3fn)
    rs_ref[...] = jnp.sum(a, axis=1, keepdims=True)


def _pass23_kernel(q_ref, rs_ref, emb_ref, e1_ref, out_ref, d8, cur, acc, cns):
    p = pl.program_id(0)
    b = pl.program_id(1)
    bb = jnp.where(p == 1, NBLK2 - 1 - b, b)
    rows = pl.ds(bb * BLK2, BLK2)

    @pl.when(b == 0)
    def _():
        x = jnp.where(p == 0, e1_ref[...], cur[...])
        c = jnp.mean(x, axis=0, keepdims=True)
        d = x - c
        s = jnp.maximum(
            jnp.max(jnp.abs(d), axis=0, keepdims=True) / F8MAX, 1e-30)
        d8[...] = (d / s).astype(jnp.float8_e4m3fn)
        cns[0:1, :] = s
        cns[1:2, :] = c

    m = jnp.dot(q_ref[...], d8[...], preferred_element_type=jnp.float32)
    val = m * cns[0:1, :] + rs_ref[...] * cns[1:2, :]

    @pl.when(p == 0)
    def _():
        cur[rows, :] = val
        acc[rows, :] = emb_ref[rows, :] + e1_ref[rows, :] + val
        out_ref[0, :, :] = val

    @pl.when(p == 1)
    def _():
        out_ref[0, :, :] = (acc[rows, :] + val) * 0.25


def kernel(interaction_matrix, user_embeds, item_embeds):
    embeds = jnp.concatenate([user_embeds, item_embeds], axis=0)
    e1, q, rowsum = pl.pallas_call(
        _pass1_kernel,
        grid=(NBLK,),
        in_specs=[
            pl.BlockSpec((BLK, N_TOTAL), lambda b: (b, 0)),
            pl.BlockSpec((N_TOTAL, EMBED), lambda b: (0, 0)),
        ],
        out_specs=[
            pl.BlockSpec((BLK, EMBED), lambda b: (b, 0)),
            pl.BlockSpec((BLK, N_TOTAL), lambda b: (b, 0)),
            pl.BlockSpec((BLK, 1), lambda b: (b, 0)),
        ],
        out_shape=[
            jax.ShapeDtypeStruct((N_TOTAL, EMBED), jnp.float32),
            jax.ShapeDtypeStruct((N_TOTAL, N_TOTAL), jnp.float8_e4m3fn),
            jax.ShapeDtypeStruct((N_TOTAL, 1), jnp.float32),
        ],
        compiler_params=pltpu.CompilerParams(
            dimension_semantics=("arbitrary",),
        ),
    )(interaction_matrix, embeds)

    out = pl.pallas_call(
        _pass23_kernel,
        grid=(2, NBLK2),
        in_specs=[
            pl.BlockSpec((BLK2, N_TOTAL),
                         lambda p, b: (jnp.where(p == 1, NBLK2 - 1 - b, b), 0)),
            pl.BlockSpec((BLK2, 1),
                         lambda p, b: (jnp.where(p == 1, NBLK2 - 1 - b, b), 0)),
            pl.BlockSpec((N_TOTAL, EMBED), lambda p, b: (0, 0)),
            pl.BlockSpec((N_TOTAL, EMBED), lambda p, b: (0, 0)),
        ],
        out_specs=pl.BlockSpec((1, BLK2, EMBED),
                               lambda p, b: (p, jnp.where(p == 1, NBLK2 - 1 - b, b), 0)),
        out_shape=jax.ShapeDtypeStruct((2, N_TOTAL, EMBED), jnp.float32),
        scratch_shapes=[
            pltpu.VMEM((N_TOTAL, EMBED), jnp.float8_e4m3fn),
            pltpu.VMEM((N_TOTAL, EMBED), jnp.float32),
            pltpu.VMEM((N_TOTAL, EMBED), jnp.float32),
            pltpu.VMEM((2, EMBED), jnp.float32),
        ],
        compiler_params=pltpu.CompilerParams(
            dimension_semantics=("arbitrary", "arbitrary"),
        ),
    )(q, rowsum, embeds, e1)

    all_emb = out[1]
    return (all_emb[:N_HALF], all_emb[N_HALF:])
